# Initial kernel scaffold; baseline (speedup 1.0000x reference)
import jax
import jax.numpy as jnp
from jax.experimental import pallas as pl


def kernel(x, y):
    raise NotImplementedError("write your pallas kernel here")


# speedup vs baseline: 633.5743x; 633.5743x over previous
import functools

import jax
import jax.numpy as jnp
from jax.experimental import pallas as pl


def _dist_block_kernel(x_ref, y_ref, out_ref):
    x = x_ref[...]
    y = y_ref[...]
    xn = jnp.sum(x * x, axis=1, keepdims=True)
    yn = jnp.sum(y * y, axis=1, keepdims=True)
    dot = jax.lax.dot_general(
        x, y, (((1,), (1,)), ((), ())),
        preferred_element_type=jnp.float32,
    )
    out_ref[...] = (xn - 2.0 * dot) + yn.T


@functools.partial(jax.jit, static_argnames=("bq", "bk"))
def _dist(x, y, bq, bk):
    q, d = x.shape
    k, _ = y.shape
    grid = (q // bq, k // bk)
    return pl.pallas_call(
        _dist_block_kernel,
        grid=grid,
        in_specs=[
            pl.BlockSpec((bq, d), lambda i, j: (i, 0)),
            pl.BlockSpec((bk, d), lambda i, j: (j, 0)),
        ],
        out_specs=pl.BlockSpec((bq, bk), lambda i, j: (i, j)),
        out_shape=jax.ShapeDtypeStruct((q, k), jnp.float32),
    )(x, y)


def kernel(x, y):
    return _dist(x, y, bq=1024, bk=2048)


# speedup vs baseline: 638.5660x; 1.0079x over previous
import functools

import jax
import jax.numpy as jnp
from jax.experimental import pallas as pl


def _dist_block_kernel(x_ref, y_ref, out_ref):
    x = x_ref[...]
    y = y_ref[...]
    xn = jnp.sum(x * x, axis=1, keepdims=True)
    yn = jnp.sum(y * y, axis=1, keepdims=True)
    dot = jax.lax.dot_general(
        x, y, (((1,), (1,)), ((), ())),
        preferred_element_type=jnp.float32,
    )
    out_ref[...] = (xn - 2.0 * dot) + yn.T


@functools.partial(jax.jit, static_argnames=("bq", "bk"))
def _dist(x, y, bq, bk):
    q, d = x.shape
    k, _ = y.shape
    grid = (q // bq, k // bk)
    return pl.pallas_call(
        _dist_block_kernel,
        grid=grid,
        in_specs=[
            pl.BlockSpec((bq, d), lambda i, j: (i, 0)),
            pl.BlockSpec((bk, d), lambda i, j: (j, 0)),
        ],
        out_specs=pl.BlockSpec((bq, bk), lambda i, j: (i, j)),
        out_shape=jax.ShapeDtypeStruct((q, k), jnp.float32),
    )(x, y)


def kernel(x, y):
    return _dist(x, y, bq=1024, bk=4096)
